# baseline (speedup 1.0000x reference)
import functools

import jax
import jax.numpy as jnp
from jax import lax
from jax.experimental import pallas as pl
from jax.experimental.pallas import tpu as pltpu
from jax.experimental.pallas import tpu_sc as plsc

_B = 16384
_D = 128
_NC = 2
_NS = 16
_NW = _NC * _NS
_BPW = _B // _NW
_CH = 128
_NCHUNK = _BPW // _CH


def _gather_body(idx_hbm, table_hbm, out_hbm, idx_v, rows_v, gsem, wsem):
    wid = lax.axis_index("s") * _NC + lax.axis_index("c")
    row0 = wid * _NCHUNK
    pltpu.sync_copy(idx_hbm.at[pl.ds(row0, _NCHUNK)], idx_v)
    for j in range(_NCHUNK):
        pltpu.async_copy(table_hbm.at[idx_v.at[j]], rows_v.at[j], gsem)
    half = _NCHUNK // 2
    for j in range(half):
        pltpu.make_async_copy(table_hbm.at[idx_v.at[j]], rows_v.at[j], gsem).wait()
    pltpu.async_copy(rows_v.at[pl.ds(0, half)], out_hbm.at[pl.ds(row0, half)], wsem)
    for j in range(half, _NCHUNK):
        pltpu.make_async_copy(table_hbm.at[idx_v.at[j]], rows_v.at[j], gsem).wait()
    pltpu.async_copy(rows_v.at[pl.ds(half, half)],
                     out_hbm.at[pl.ds(row0 + half, half)], wsem)
    pltpu.make_async_copy(rows_v.at[pl.ds(0, half)],
                          out_hbm.at[pl.ds(row0, half)], wsem).wait()
    pltpu.make_async_copy(rows_v.at[pl.ds(half, half)],
                          out_hbm.at[pl.ds(row0 + half, half)], wsem).wait()


@jax.jit
def _run(labels2d, embeddings):
    mesh = plsc.VectorSubcoreMesh(core_axis_name="c", subcore_axis_name="s")
    fn = functools.partial(
        pl.kernel,
        out_type=jax.ShapeDtypeStruct((_B // _CH, _CH, _D), jnp.float32),
        mesh=mesh,
        scratch_types=[
            pltpu.VMEM((_NCHUNK, _CH), jnp.int32),
            pltpu.VMEM((_NCHUNK, _CH, _D), jnp.float32),
            pltpu.SemaphoreType.DMA,
            pltpu.SemaphoreType.DMA,
        ],
    )(_gather_body)
    return fn(labels2d, embeddings)


def kernel(labels, embeddings):
    labels2d = labels.astype(jnp.int32).reshape(_B // _CH, _CH)
    out = _run(labels2d, embeddings)
    return out.reshape(_B, _D)


# speedup vs baseline: 1.0231x; 1.0231x over previous
import functools

import jax
import jax.numpy as jnp
from jax import lax
from jax.experimental import pallas as pl
from jax.experimental.pallas import tpu as pltpu
from jax.experimental.pallas import tpu_sc as plsc

_B = 16384
_D = 128
_NC = 2
_NS = 16
_NW = _NC * _NS
_BPW = _B // _NW
_CH = 512
_NCHUNK = _BPW // _CH


def _gather_body(idx_hbm, table_hbm, out_hbm, idx_v, rows_v, gsem, wsem):
    wid = lax.axis_index("s") * _NC + lax.axis_index("c")
    row0 = wid * _NCHUNK
    pltpu.sync_copy(idx_hbm.at[pl.ds(row0, _NCHUNK)], idx_v)
    for j in range(_NCHUNK):
        pltpu.async_copy(table_hbm.at[idx_v.at[j]], rows_v.at[j], gsem)
    for j in range(_NCHUNK):
        pltpu.make_async_copy(table_hbm.at[idx_v.at[j]], rows_v.at[j], gsem).wait()
    pltpu.sync_copy(rows_v, out_hbm.at[pl.ds(row0, _NCHUNK)])


@jax.jit
def _run(labels2d, embeddings):
    mesh = plsc.VectorSubcoreMesh(core_axis_name="c", subcore_axis_name="s")
    fn = functools.partial(
        pl.kernel,
        out_type=jax.ShapeDtypeStruct((_B // _CH, _CH, _D), jnp.float32),
        mesh=mesh,
        scratch_types=[
            pltpu.VMEM((_NCHUNK, _CH), jnp.int32),
            pltpu.VMEM((_NCHUNK, _CH, _D), jnp.float32),
            pltpu.SemaphoreType.DMA,
            pltpu.SemaphoreType.DMA,
        ],
    )(_gather_body)
    return fn(labels2d, embeddings)


def kernel(labels, embeddings):
    labels2d = labels.astype(jnp.int32).reshape(_B // _CH, _CH)
    out = _run(labels2d, embeddings)
    return out.reshape(_B, _D)
